# baseline (speedup 1.0000x reference)
import functools

import jax
import jax.numpy as jnp
import numpy as np
from jax import lax
from jax.experimental import pallas as pl
from jax.experimental.pallas import tpu as pltpu
from jax.experimental.pallas import tpu_sc as plsc

_B = 16384
_D = 128
_NC = 2
_NS = 16
_NW = _NC * _NS
_BPW = _B // _NW

_GC = 256
_PC = _GC // 4


_U32 = np.uint32


def _rotl(x, r):
    r = _U32(r)
    return (x << r) | (x >> _U32(32 - r))


def _threefry2x32(k1, k2, x0, x1):
    with np.errstate(over="ignore"):
        rot = [np.array([13, 15, 26, 6], np.uint32),
               np.array([17, 29, 16, 24], np.uint32)]
        ks = [_U32(k1), _U32(k2), _U32(k1) ^ _U32(k2) ^ _U32(0x1BD11BDA)]
        x = [x0.astype(np.uint32) + ks[0], x1.astype(np.uint32) + ks[1]]

        def rounds(x, rs):
            for r in rs:
                a = x[0] + x[1]
                b = _rotl(x[1], r)
                x = [a, a ^ b]
            return x

        x = rounds(x, rot[0])
        x = [x[0] + ks[1], x[1] + ks[2] + _U32(1)]
        x = rounds(x, rot[1])
        x = [x[0] + ks[2], x[1] + ks[0] + _U32(2)]
        x = rounds(x, rot[0])
        x = [x[0] + ks[0], x[1] + ks[1] + _U32(3)]
        x = rounds(x, rot[1])
        x = [x[0] + ks[1], x[1] + ks[2] + _U32(4)]
        x = rounds(x, rot[0])
        x = [x[0] + ks[2], x[1] + ks[0] + _U32(5)]
    return x


def _split2(k1, k2):
    b1, b2 = _threefry2x32(k1, k2, np.zeros(2, np.uint32),
                           np.arange(2, dtype=np.uint32))
    return (b1[0], b2[0]), (b1[1], b2[1])


def _random_bits32(key, n):
    b1, b2 = _threefry2x32(key[0], key[1], np.zeros(n, np.uint32),
                           np.arange(n, dtype=np.uint32))
    return b1 ^ b2


def _randint_np(key, n, span_int):
    with np.errstate(over="ignore"):
        k1, k2 = _split2(*key)
        hi = _random_bits32(k1, n)
        lo = _random_bits32(k2, n)
        span = _U32(span_int)
        mult = _U32(2 ** 16) % span
        mult = (mult * mult) % span
        off = ((hi % span) * mult + (lo % span)) % span
    return off.astype(np.int32)


def _normal_np(key, n):
    from scipy.special import erfinv
    bits = _random_bits32(key, n)
    fb = (bits >> _U32(9)) | np.float32(1.0).view(np.uint32)
    floats = fb.view(np.float32) - np.float32(1.0)
    lo = np.nextafter(np.float32(-1.0), np.float32(0.0), dtype=np.float32)
    u = np.maximum(lo, (floats * (np.float32(1.0) - lo) + lo).astype(np.float32))
    return (np.float32(np.sqrt(2)) * erfinv(u.astype(np.float64))).astype(np.float32)


def _fixed_draws():
    key = (_U32(0), _U32(42))
    k1, k2 = _split2(*key)
    neg = _randint_np(k1, _B * 4, 100000)
    z = _normal_np(k2, _B * _D).reshape(_B, _D)
    delta = np.clip(z * np.float32(0.1) + np.float32(0.5),
                    np.float32(0.0), np.float32(1.0)).astype(np.float32)
    return neg, delta


_NEG_FLAT_NP, _DELTA_NP = _fixed_draws()


def _mix64(g, v, dl, m):
    def body_fn(r, carry):
        r4 = 4 * r
        for sl16 in range(_D // 16):
            sl = pl.ds(sl16 * 16, 16)
            pot = (g[r4, sl] + g[r4 + 1, sl]
                   + g[r4 + 2, sl] + g[r4 + 3, sl]) * 0.25
            m[r, sl] = pot + dl[r, sl] * (v[r, sl] - pot)
        return carry
    lax.fori_loop(0, _PC, body_fn, 0)


_NBUF = 3


def _run_pipeline(jobs, base, gbufs, gsems, wsems):
    def start_gather(k):
        p = k % _NBUF
        _, h, idx_ref, table, _ = jobs[k]
        return pltpu.async_copy(
            table.at[idx_ref.at[pl.ds(h * _GC, _GC)]], gbufs[p], gsems[p])

    n = len(jobs)
    wpend = [None] * _NBUF
    hg = [None] * _NBUF
    for k in range(min(_NBUF - 1, n)):
        hg[k % _NBUF] = start_gather(k)
    for k in range(n):
        p = k % _NBUF
        if k + _NBUF - 1 < n:
            q = (k + _NBUF - 1) % _NBUF
            if wpend[q] is not None:
                wpend[q].wait()
                wpend[q] = None
            hg[q] = start_gather(k + _NBUF - 1)
        hg[p].wait()
        _, h, _, _, out = jobs[k]
        wpend[p] = pltpu.async_copy(
            gbufs[p], out.at[pl.ds(base + h * _GC, _GC)], wsems[p])
    for hnd in wpend:
        if hnd is not None:
            hnd.wait()


def _sc1_body(u_emb, v_emb, idx_u, idx_v, neg_flat, delta,
              u_id_out, vmix_out,
              ixu, ixv, ixn, g0, g1, vb0, vb1, db0, db1, mb0, mb1,
              sgN0, sgN1, sgV0, sgV1, sgD0, sgD1, swM0, swM1, swU0, swU1):
    c = lax.axis_index("c")
    s = lax.axis_index("s")
    wid = s * _NC + c
    base = wid * _BPW
    pltpu.sync_copy(idx_u.at[pl.ds(base, _BPW)], ixu)
    pltpu.sync_copy(idx_v.at[pl.ds(base, _BPW)], ixv)
    pltpu.sync_copy(neg_flat.at[pl.ds(4 * base, 4 * _BPW)], ixn)
    gbufs = (g0, g1)
    vbufs = (vb0, vb1)
    dbufs = (db0, db1)
    mbufs = (mb0, mb1)
    sgN = (sgN0, sgN1)
    sgV = (sgV0, sgV1)
    sgD = (sgD0, sgD1)
    swM = (swM0, swM1)
    swU = (swU0, swU1)
    jobs = [("mix", i) for i in range(8)] + [("u", 0), ("u", 1)]

    def start(k):
        p = k % 2
        job = jobs[k]
        if job[0] == "mix":
            i = job[1]
            hN = pltpu.async_copy(
                v_emb.at[ixn.at[pl.ds(i * _GC, _GC)]], gbufs[p], sgN[p])
            hV = pltpu.async_copy(
                v_emb.at[ixv.at[pl.ds(i * _PC, _PC)]], vbufs[p], sgV[p])
            hD = pltpu.async_copy(
                delta.at[pl.ds(base + i * _PC, _PC)], dbufs[p], sgD[p])
            return (hN, hV, hD)
        j = job[1]
        return (pltpu.async_copy(
            u_emb.at[ixu.at[pl.ds(j * _GC, _GC)]], gbufs[p], sgN[p]),)

    n = len(jobs)
    wpend = [None, None]
    mpend = [None, None]
    hg = [None, None]
    hg[0] = start(0)
    for k in range(n):
        p = k % 2
        if k + 1 < n:
            q = (k + 1) % 2
            if wpend[q] is not None:
                wpend[q].wait()
                wpend[q] = None
            hg[q] = start(k + 1)
        for hnd in hg[p]:
            hnd.wait()
        job = jobs[k]
        if job[0] == "mix":
            i = job[1]
            if mpend[p] is not None:
                mpend[p].wait()
                mpend[p] = None
            _mix64(gbufs[p], vbufs[p], dbufs[p], mbufs[p])
            mpend[p] = pltpu.async_copy(
                mbufs[p], vmix_out.at[pl.ds(base + i * _PC, _PC)], swM[p])
        else:
            j = job[1]
            wpend[p] = pltpu.async_copy(
                gbufs[p], u_id_out.at[pl.ds(base + j * _GC, _GC)], swU[p])
    for hnd in (*wpend, *mpend):
        if hnd is not None:
            hnd.wait()


def _sc2_body(u_rev, v_rev, idx_u, idx_v,
              u_rev_out, v_rev_out,
              ixu, ixv, g0, g1, g2, sg0, sg1, sg2, sw0, sw1, sw2):
    c = lax.axis_index("c")
    s = lax.axis_index("s")
    wid = s * _NC + c
    base = wid * _BPW
    pltpu.sync_copy(idx_u.at[pl.ds(base, _BPW)], ixu)
    pltpu.sync_copy(idx_v.at[pl.ds(base, _BPW)], ixv)
    jobs = [("main", h, ib, t, o)
            for (ib, t, o) in ((ixu, u_rev, u_rev_out), (ixv, v_rev, v_rev_out))
            for h in range(2)]
    _run_pipeline(jobs, base, (g0, g1, g2), (sg0, sg1, sg2), (sw0, sw1, sw2))


@functools.lru_cache(maxsize=None)
def _get_sc_calls():
    mesh = plsc.VectorSubcoreMesh(
        core_axis_name="c", subcore_axis_name="s",
        num_cores=_NC, num_subcores=_NS)
    sc1 = pl.kernel(
        _sc1_body,
        out_type=[jax.ShapeDtypeStruct((_B, _D), jnp.float32)] * 2,
        mesh=mesh,
        scratch_types=[
            pltpu.VMEM((_BPW,), jnp.int32),
            pltpu.VMEM((_BPW,), jnp.int32),
            pltpu.VMEM((4 * _BPW,), jnp.int32),
            pltpu.VMEM((_GC, _D), jnp.float32),
            pltpu.VMEM((_GC, _D), jnp.float32),
            pltpu.VMEM((_PC, _D), jnp.float32),
            pltpu.VMEM((_PC, _D), jnp.float32),
            pltpu.VMEM((_PC, _D), jnp.float32),
            pltpu.VMEM((_PC, _D), jnp.float32),
            pltpu.VMEM((_PC, _D), jnp.float32),
            pltpu.VMEM((_PC, _D), jnp.float32),
        ] + [pltpu.SemaphoreType.DMA] * 10,
    )
    sc2 = pl.kernel(
        _sc2_body,
        out_type=[jax.ShapeDtypeStruct((_B, _D), jnp.float32)] * 2,
        mesh=mesh,
        scratch_types=[
            pltpu.VMEM((_BPW,), jnp.int32),
            pltpu.VMEM((_BPW,), jnp.int32),
            pltpu.VMEM((_GC, _D), jnp.float32),
            pltpu.VMEM((_GC, _D), jnp.float32),
            pltpu.VMEM((_GC, _D), jnp.float32),
        ] + [pltpu.SemaphoreType.DMA] * 6,
    )
    return sc1, sc2


_T1 = 4
_RT1 = _B // _T1
_EPS = 1e-5
_DN = (((1,), (1,)), ((), ()))


def _bn_full(z, g, b):
    mu = jnp.mean(z, axis=0, keepdims=True)
    zc = z - mu
    var = jnp.mean(zc * zc, axis=0, keepdims=True)
    return g * (zc * lax.rsqrt(var + _EPS)) + b


def _tc_body(u_id, vmix_in, W1, b1, W2, b2, W3, b3, Wc, bc,
             g1, be1, g2, be2, g3, be3,
             pred_o, z1s, s1):
    i = pl.program_id(0)
    f32 = jnp.float32

    @pl.when(i < _T1)
    def _():
        w1 = W1[...]
        a1 = (lax.dot_general(u_id[...], w1[:, :_D], _DN,
                              preferred_element_type=f32)
              + lax.dot_general(vmix_in[...], w1[:, _D:], _DN,
                                preferred_element_type=f32))
        z1 = jnp.maximum(a1 + b1[...], 0.0)
        z1s[pl.ds(i * _RT1, _RT1), :] = z1
        st = jnp.concatenate([jnp.sum(z1, axis=0, keepdims=True),
                              jnp.sum(z1 * z1, axis=0, keepdims=True)], axis=0)

        @pl.when(i == 0)
        def _():
            s1[...] = st

        @pl.when(i > 0)
        def _():
            s1[...] = s1[...] + st

    @pl.when(i == _T1)
    def _():
        inv_b = 1.0 / _B
        st = s1[...]
        mu = st[0:1, :] * inv_b
        var = st[1:2, :] * inv_b - mu * mu
        h1 = g1[...] * ((z1s[...] - mu) * lax.rsqrt(var + _EPS)) + be1[...]
        z2 = jnp.maximum(
            lax.dot_general(h1, W2[...], _DN, preferred_element_type=f32)
            + b2[...], 0.0)
        h2 = _bn_full(z2, g2[...], be2[...])
        z3 = jnp.maximum(
            lax.dot_general(h2, W3[...], _DN, preferred_element_type=f32)
            + b3[...], 0.0)
        h3 = _bn_full(z3, g3[...], be3[...])
        logit = lax.dot_general(Wc[...], h3, _DN,
                                preferred_element_type=f32) + bc[...]
        pred_o[...] = jnp.reshape(1.0 / (1.0 + jnp.exp(-logit)), (_B,))


def _row_spec(n):
    return pl.BlockSpec((_RT1, n), lambda i: (jnp.minimum(i, _T1 - 1), 0))


def _const_spec(m, n):
    return pl.BlockSpec((m, n), lambda i: (0, 0))


_tc = pl.pallas_call(
    _tc_body,
    grid=(_T1 + 1,),
    in_specs=[_row_spec(_D), _row_spec(_D),
              _const_spec(_D, 2 * _D), _const_spec(1, _D),
              _const_spec(_D // 2, _D), _const_spec(1, _D // 2),
              _const_spec(_D // 4, _D // 2), _const_spec(1, _D // 4),
              _const_spec(1, _D // 4), _const_spec(1, 1),
              _const_spec(1, _D), _const_spec(1, _D),
              _const_spec(1, _D // 2), _const_spec(1, _D // 2),
              _const_spec(1, _D // 4), _const_spec(1, _D // 4)],
    out_specs=pl.BlockSpec((_B,), lambda i: (0,)),
    out_shape=jax.ShapeDtypeStruct((_B,), jnp.float32),
    scratch_shapes=[pltpu.VMEM((_B, _D), jnp.float32),
                    pltpu.VMEM((2, _D), jnp.float32)],
)


def kernel(nodes_u, nodes_v, global_protos, inter_nums, u_emb_w, v_emb_w,
           u_rev_w, v_rev_w, W1, b1, W2, b2, W3, b3, Wc, bc,
           g1, be1, g2, be2, g3, be3):
    nB = nodes_u.shape[0]
    delta = jnp.asarray(_DELTA_NP)
    neg_flat = jnp.asarray(_NEG_FLAT_NP)
    idx_u = nodes_u.astype(jnp.int32)
    idx_v = nodes_v.astype(jnp.int32)

    sc1, sc2 = _get_sc_calls()
    u_id, vmix = sc1(u_emb_w, v_emb_w, idx_u, idx_v, neg_flat, delta)
    u_rev, v_rev = sc2(u_rev_w, v_rev_w, idx_u, idx_v)

    r = lambda a: a.reshape(1, -1)
    pred = _tc(u_id, vmix,
               W1, r(b1), W2, r(b2), W3, r(b3), Wc, bc.reshape(1, 1),
               r(g1), r(be1), r(g2), r(be2), r(g3), r(be3))
    return (pred, u_id, vmix, u_rev, v_rev)


# speedup vs baseline: 1.1258x; 1.1258x over previous
import functools

import jax
import jax.numpy as jnp
import numpy as np
from jax import lax
from jax.experimental import pallas as pl
from jax.experimental.pallas import tpu as pltpu
from jax.experimental.pallas import tpu_sc as plsc

_B = 16384
_D = 128
_NC = 2
_NS = 16
_NW = _NC * _NS
_BPW = _B // _NW

_GC = 256
_PC = _GC // 4


_U32 = np.uint32


def _rotl(x, r):
    r = _U32(r)
    return (x << r) | (x >> _U32(32 - r))


def _threefry2x32(k1, k2, x0, x1):
    with np.errstate(over="ignore"):
        rot = [np.array([13, 15, 26, 6], np.uint32),
               np.array([17, 29, 16, 24], np.uint32)]
        ks = [_U32(k1), _U32(k2), _U32(k1) ^ _U32(k2) ^ _U32(0x1BD11BDA)]
        x = [x0.astype(np.uint32) + ks[0], x1.astype(np.uint32) + ks[1]]

        def rounds(x, rs):
            for r in rs:
                a = x[0] + x[1]
                b = _rotl(x[1], r)
                x = [a, a ^ b]
            return x

        x = rounds(x, rot[0])
        x = [x[0] + ks[1], x[1] + ks[2] + _U32(1)]
        x = rounds(x, rot[1])
        x = [x[0] + ks[2], x[1] + ks[0] + _U32(2)]
        x = rounds(x, rot[0])
        x = [x[0] + ks[0], x[1] + ks[1] + _U32(3)]
        x = rounds(x, rot[1])
        x = [x[0] + ks[1], x[1] + ks[2] + _U32(4)]
        x = rounds(x, rot[0])
        x = [x[0] + ks[2], x[1] + ks[0] + _U32(5)]
    return x


def _split2(k1, k2):
    b1, b2 = _threefry2x32(k1, k2, np.zeros(2, np.uint32),
                           np.arange(2, dtype=np.uint32))
    return (b1[0], b2[0]), (b1[1], b2[1])


def _random_bits32(key, n):
    b1, b2 = _threefry2x32(key[0], key[1], np.zeros(n, np.uint32),
                           np.arange(n, dtype=np.uint32))
    return b1 ^ b2


def _randint_np(key, n, span_int):
    with np.errstate(over="ignore"):
        k1, k2 = _split2(*key)
        hi = _random_bits32(k1, n)
        lo = _random_bits32(k2, n)
        span = _U32(span_int)
        mult = _U32(2 ** 16) % span
        mult = (mult * mult) % span
        off = ((hi % span) * mult + (lo % span)) % span
    return off.astype(np.int32)


def _normal_np(key, n):
    from scipy.special import erfinv
    bits = _random_bits32(key, n)
    fb = (bits >> _U32(9)) | np.float32(1.0).view(np.uint32)
    floats = fb.view(np.float32) - np.float32(1.0)
    lo = np.nextafter(np.float32(-1.0), np.float32(0.0), dtype=np.float32)
    u = np.maximum(lo, (floats * (np.float32(1.0) - lo) + lo).astype(np.float32))
    return (np.float32(np.sqrt(2)) * erfinv(u.astype(np.float64))).astype(np.float32)


def _fixed_draws():
    key = (_U32(0), _U32(42))
    k1, k2 = _split2(*key)
    neg = _randint_np(k1, _B * 4, 100000)
    z = _normal_np(k2, _B * _D).reshape(_B, _D)
    delta = np.clip(z * np.float32(0.1) + np.float32(0.5),
                    np.float32(0.0), np.float32(1.0)).astype(np.float32)
    return neg, delta


_NEG_FLAT_NP, _DELTA_NP = _fixed_draws()
_DELTA_BF16_NP = _DELTA_NP.astype(jnp.bfloat16)


def _pool4(g, p):
    def body_fn(r, carry):
        r4 = 4 * r
        for sl16 in range(_D // 16):
            sl = pl.ds(sl16 * 16, 16)
            p[r, sl] = g[r4, sl] + g[r4 + 1, sl] + g[r4 + 2, sl] + g[r4 + 3, sl]
        return carry
    lax.fori_loop(0, _PC, body_fn, 0)


_NBUF = 3


def _run_pipeline(jobs, base, pot_out, gbufs, gsems, wsems, pbufs, psem):
    def start_gather(k):
        p = k % _NBUF
        job = jobs[k]
        if job[0] == "neg":
            _, i, idx_ref, table = job
            return pltpu.async_copy(
                table.at[idx_ref.at[pl.ds(i * _GC, _GC)]], gbufs[p], gsems[p])
        _, h, idx_ref, table, _ = job
        return pltpu.async_copy(
            table.at[idx_ref.at[pl.ds(h * _GC, _GC)]], gbufs[p], gsems[p])

    n = len(jobs)
    wpend = [None] * _NBUF
    ppend = [None, None]
    hg = [None] * _NBUF
    for k in range(min(_NBUF - 1, n)):
        hg[k % _NBUF] = start_gather(k)
    for k in range(n):
        p = k % _NBUF
        if k + _NBUF - 1 < n:
            q = (k + _NBUF - 1) % _NBUF
            if wpend[q] is not None:
                wpend[q].wait()
                wpend[q] = None
            hg[q] = start_gather(k + _NBUF - 1)
        hg[p].wait()
        job = jobs[k]
        if job[0] == "neg":
            i = job[1]
            pp = i % 2
            if ppend[pp] is not None:
                ppend[pp].wait()
                ppend[pp] = None
            _pool4(gbufs[p], pbufs[pp])
            ppend[pp] = pltpu.async_copy(
                pbufs[pp], pot_out.at[pl.ds(base + i * _PC, _PC)], psem)
        else:
            _, h, _, _, out = job
            wpend[p] = pltpu.async_copy(
                gbufs[p], out.at[pl.ds(base + h * _GC, _GC)], wsems[p])
    for hnd in (*wpend, *ppend):
        if hnd is not None:
            hnd.wait()


def _sc1_body(u_emb, v_emb, idx_all,
              u_id_out, v_id_out, pot_out,
              ixu, ixv, ixn, g0, g1, g2, p0, p1,
              sg0, sg1, sg2, sw0, sw1, sw2, swp):
    c = lax.axis_index("c")
    s = lax.axis_index("s")
    wid = s * _NC + c
    base = wid * _BPW
    pltpu.sync_copy(idx_all.at[pl.ds(base, _BPW)], ixu)
    pltpu.sync_copy(idx_all.at[pl.ds(_B + base, _BPW)], ixv)
    pltpu.sync_copy(idx_all.at[pl.ds(2 * _B + 4 * base, 4 * _BPW)], ixn)
    mains = [("main", h, ib, t, o)
             for (ib, t, o) in ((ixu, u_emb, u_id_out), (ixv, v_emb, v_id_out))
             for h in range(2)]
    negs = [("neg", i, ixn, v_emb) for i in range(8)]
    jobs = []
    for i in range(8):
        jobs.append(negs[i])
        if i < 4:
            jobs.append(mains[i])
    _run_pipeline(jobs, base, pot_out, (g0, g1, g2), (sg0, sg1, sg2),
                  (sw0, sw1, sw2), (p0, p1), swp)


def _sc2_body(u_rev, v_rev, idx_all,
              u_rev_out, v_rev_out,
              ixu, ixv, g0, g1, g2, sg0, sg1, sg2, sw0, sw1, sw2):
    c = lax.axis_index("c")
    s = lax.axis_index("s")
    wid = s * _NC + c
    base = wid * _BPW
    pltpu.sync_copy(idx_all.at[pl.ds(base, _BPW)], ixu)
    pltpu.sync_copy(idx_all.at[pl.ds(_B + base, _BPW)], ixv)
    jobs = [("main", h, ib, t, o)
            for (ib, t, o) in ((ixu, u_rev, u_rev_out), (ixv, v_rev, v_rev_out))
            for h in range(2)]
    _run_pipeline(jobs, base, None, (g0, g1, g2), (sg0, sg1, sg2),
                  (sw0, sw1, sw2), (None, None), None)


@functools.lru_cache(maxsize=None)
def _get_sc_calls():
    mesh = plsc.VectorSubcoreMesh(
        core_axis_name="c", subcore_axis_name="s",
        num_cores=_NC, num_subcores=_NS)
    sc1 = pl.kernel(
        _sc1_body,
        out_type=[jax.ShapeDtypeStruct((_B, _D), jnp.float32)] * 3,
        mesh=mesh,
        scratch_types=[
            pltpu.VMEM((_BPW,), jnp.int32),
            pltpu.VMEM((_BPW,), jnp.int32),
            pltpu.VMEM((4 * _BPW,), jnp.int32),
            pltpu.VMEM((_GC, _D), jnp.float32),
            pltpu.VMEM((_GC, _D), jnp.float32),
            pltpu.VMEM((_GC, _D), jnp.float32),
            pltpu.VMEM((_PC, _D), jnp.float32),
            pltpu.VMEM((_PC, _D), jnp.float32),
        ] + [pltpu.SemaphoreType.DMA] * 7,
    )
    sc2 = pl.kernel(
        _sc2_body,
        out_type=[jax.ShapeDtypeStruct((_B, _D), jnp.float32)] * 2,
        mesh=mesh,
        scratch_types=[
            pltpu.VMEM((_BPW,), jnp.int32),
            pltpu.VMEM((_BPW,), jnp.int32),
            pltpu.VMEM((_GC, _D), jnp.float32),
            pltpu.VMEM((_GC, _D), jnp.float32),
            pltpu.VMEM((_GC, _D), jnp.float32),
        ] + [pltpu.SemaphoreType.DMA] * 6,
    )
    return sc1, sc2


_T1 = 4
_RT1 = _B // _T1
_EPS = 1e-5
_DN = (((1,), (1,)), ((), ()))


def _bn_full(z, g, b):
    mu = jnp.mean(z, axis=0, keepdims=True)
    zc = z - mu
    var = jnp.mean(zc * zc, axis=0, keepdims=True)
    return g * (zc * lax.rsqrt(var + _EPS)) + b


def _tc_body(u_id, v_raw, pot_s, delta, W1, b1, W2, b2, W3, b3, Wc, bc,
             g1, be1, g2, be2, g3, be3,
             vmix_o, pred_o, z1s, s1):
    i = pl.program_id(0)
    f32 = jnp.float32

    @pl.when(i < _T1)
    def _():
        d = delta[...].astype(f32)
        vmix = d * v_raw[...] + (1.0 - d) * (pot_s[...] * 0.25)
        vmix_o[...] = vmix
        w1 = W1[...]
        a1 = (lax.dot_general(u_id[...], w1[:, :_D], _DN,
                              preferred_element_type=f32)
              + lax.dot_general(vmix, w1[:, _D:], _DN,
                                preferred_element_type=f32))
        z1 = jnp.maximum(a1 + b1[...], 0.0)
        z1s[pl.ds(i * _RT1, _RT1), :] = z1
        st = jnp.concatenate([jnp.sum(z1, axis=0, keepdims=True),
                              jnp.sum(z1 * z1, axis=0, keepdims=True)], axis=0)

        @pl.when(i == 0)
        def _():
            s1[...] = st

        @pl.when(i > 0)
        def _():
            s1[...] = s1[...] + st

    @pl.when(i == _T1)
    def _():
        inv_b = 1.0 / _B
        st = s1[...]
        mu = st[0:1, :] * inv_b
        var = st[1:2, :] * inv_b - mu * mu
        h1 = g1[...] * ((z1s[...] - mu) * lax.rsqrt(var + _EPS)) + be1[...]
        z2 = jnp.maximum(
            lax.dot_general(h1, W2[...], _DN, preferred_element_type=f32)
            + b2[...], 0.0)
        h2 = _bn_full(z2, g2[...], be2[...])
        z3 = jnp.maximum(
            lax.dot_general(h2, W3[...], _DN, preferred_element_type=f32)
            + b3[...], 0.0)
        h3 = _bn_full(z3, g3[...], be3[...])
        logit = lax.dot_general(Wc[...], h3, _DN,
                                preferred_element_type=f32) + bc[...]
        pred_o[...] = jnp.reshape(1.0 / (1.0 + jnp.exp(-logit)), (_B,))


def _row_spec(n):
    return pl.BlockSpec((_RT1, n), lambda i: (jnp.minimum(i, _T1 - 1), 0))


def _const_spec(m, n):
    return pl.BlockSpec((m, n), lambda i: (0, 0))


_tc = pl.pallas_call(
    _tc_body,
    grid=(_T1 + 1,),
    in_specs=[_row_spec(_D), _row_spec(_D), _row_spec(_D), _row_spec(_D),
              _const_spec(_D, 2 * _D), _const_spec(1, _D),
              _const_spec(_D // 2, _D), _const_spec(1, _D // 2),
              _const_spec(_D // 4, _D // 2), _const_spec(1, _D // 4),
              _const_spec(1, _D // 4), _const_spec(1, 1),
              _const_spec(1, _D), _const_spec(1, _D),
              _const_spec(1, _D // 2), _const_spec(1, _D // 2),
              _const_spec(1, _D // 4), _const_spec(1, _D // 4)],
    out_specs=(_row_spec(_D), pl.BlockSpec((_B,), lambda i: (0,))),
    out_shape=(jax.ShapeDtypeStruct((_B, _D), jnp.float32),
               jax.ShapeDtypeStruct((_B,), jnp.float32)),
    scratch_shapes=[pltpu.VMEM((_B, _D), jnp.float32),
                    pltpu.VMEM((2, _D), jnp.float32)],
)


def kernel(nodes_u, nodes_v, global_protos, inter_nums, u_emb_w, v_emb_w,
           u_rev_w, v_rev_w, W1, b1, W2, b2, W3, b3, Wc, bc,
           g1, be1, g2, be2, g3, be3):
    nB = nodes_u.shape[0]
    delta = jnp.asarray(_DELTA_BF16_NP)
    idx_all = jnp.concatenate([nodes_u.astype(jnp.int32),
                               nodes_v.astype(jnp.int32),
                               jnp.asarray(_NEG_FLAT_NP)])

    sc1, sc2 = _get_sc_calls()
    u_id, v_id_raw, pot_sum = sc1(u_emb_w, v_emb_w, idx_all)
    u_rev, v_rev = sc2(u_rev_w, v_rev_w, idx_all)

    r = lambda a: a.reshape(1, -1)
    vmix, pred = _tc(u_id, v_id_raw, pot_sum, delta,
                     W1, r(b1), W2, r(b2), W3, r(b3), Wc, bc.reshape(1, 1),
                     r(g1), r(be1), r(g2), r(be2), r(g3), r(be3))
    return (pred, u_id, vmix, u_rev, v_rev)
